# baseline (speedup 1.0000x reference)
import functools

import jax
import jax.numpy as jnp
from jax import lax
from jax.experimental import pallas as pl
from jax.experimental.pallas import tpu as pltpu
from jax.experimental.pallas import tpu_sc as plsc

B = 16384
F = 26
V = 1000
D = 128
NCF = 4
ROWS = B * F
NPART = 4
PB = B // NPART
LOG2PB = 12
PROWS = PB * F

SC_CORES = 2
SC_SUBCORES = 16
NW = SC_CORES * SC_SUBCORES
PROWS_PER_W = PROWS // NW
CH = 256
CHB = CH // 128
PNCHUNK = PROWS_PER_W // CH


def _sc_gather_part(fitp, emb):
    mesh = plsc.VectorSubcoreMesh(core_axis_name="c", subcore_axis_name="s")

    @functools.partial(
        pl.kernel,
        mesh=mesh,
        out_type=jax.ShapeDtypeStruct((PROWS, D), jnp.float32),
        scratch_types=[
            pltpu.VMEM((CHB, 128), jnp.int32),
            pltpu.VMEM((CH, D), jnp.float32),
            pltpu.SemaphoreType.DMA,
        ],
    )
    def k(fi_hbm, emb_hbm, oute_hbm, idx_v, rows_v, sem_e):
        wid = lax.axis_index("c") * SC_SUBCORES + lax.axis_index("s")
        w_base = wid * PROWS_PER_W
        w_row0 = wid * (PROWS_PER_W // 128)

        def chunk_body(ci, carry):
            qp0 = w_base + ci * CH
            fld = lax.shift_right_logical(qp0, LOG2PB)
            rb = w_row0 + ci * CHB
            pltpu.sync_copy(fi_hbm.at[pl.ds(rb, CHB)], idx_v)
            off = fld * V
            for j in range(CHB):
                for k16 in range(8):
                    sl = (j, pl.ds(k16 * 16, 16))
                    idx_v[sl] = idx_v[sl] + off
            copies = [
                pltpu.make_async_copy(
                    emb_hbm.at[idx_v.at[j]],
                    rows_v.at[pl.ds(j * 128, 128)], sem_e)
                for j in range(CHB)
            ]
            for c in copies:
                c.start()
            for c in copies:
                c.wait()
            pltpu.sync_copy(rows_v, oute_hbm.at[pl.ds(qp0, CH)])
            return carry

        lax.fori_loop(0, PNCHUNK, chunk_body, 0)

    return k(fitp, emb)


LCH = 256
LNCHUNK = PROWS_PER_W // LCH


def _sc_linear_part(fi_flat, lin_flat):
    mesh = plsc.VectorSubcoreMesh(core_axis_name="c", subcore_axis_name="s")

    @functools.partial(
        pl.kernel,
        mesh=mesh,
        out_type=jax.ShapeDtypeStruct((F, PB), jnp.float32),
        scratch_types=[
            pltpu.VMEM((F * V,), jnp.float32),
            pltpu.VMEM((LCH,), jnp.int32),
            pltpu.VMEM((LCH,), jnp.float32),
        ],
        compiler_params=pltpu.CompilerParams(needs_layout_passes=False),
    )
    def k(fi_hbm, lin_hbm, outl_hbm, tab_v, idx_v, val_v):
        wid = lax.axis_index("c") * SC_SUBCORES + lax.axis_index("s")
        w_base = wid * PROWS_PER_W
        pltpu.sync_copy(lin_hbm, tab_v)

        def chunk_body(ci, carry):
            base = w_base + ci * LCH
            fld = lax.shift_right_logical(base, LOG2PB)
            s0 = base - fld * PB
            pltpu.sync_copy(fi_hbm.at[pl.ds(base, LCH)], idx_v)
            off = fld * V
            for j in range(LCH // 16):
                sl = pl.ds(j * 16, 16)
                val_v[sl] = plsc.load_gather(tab_v, [idx_v[sl] + off])
            pltpu.sync_copy(val_v, outl_hbm.at[fld, pl.ds(s0, LCH)])
            return carry

        lax.fori_loop(0, LNCHUNK, chunk_body, 0)

    return k(fi_flat, lin_flat)


def _ln(h, g, b):
    m = jnp.mean(h, axis=-1, keepdims=True)
    d = h - m
    v = jnp.mean(d * d, axis=-1, keepdims=True)
    return d * lax.rsqrt(v + 1e-5) * g[None, :] + b[None, :]


def _make_tc_body(with_prev):
    def body(x3_ref, cont_ref, lv_ref, w1e_ref, w1c_ref, b1_ref, g1_ref,
             be1_ref, w2_ref, b2_ref, g2_ref, be2_ref, w3_ref, b3_ref,
             g3_ref, be3_ref, w4_ref, b4_ref, *rest):
        if with_prev:
            _, out_ref, embout_ref = rest
        else:
            out_ref, embout_ref = rest
        x3 = x3_ref[...]
        embout_ref[...] = x3
        xf = x3[0]
        s = xf
        ss = xf * xf
        for f in range(1, F):
            xf = x3[f]
            s = s + xf
            ss = ss + xf * xf
        xall = jnp.concatenate([x3[f] for f in range(F)], axis=1)
        h = jnp.dot(xall.astype(jnp.bfloat16), w1e_ref[...],
                    preferred_element_type=jnp.float32)
        second = 0.5 * jnp.sum(s * s - ss, axis=1, keepdims=True)
        first = jnp.sum(lv_ref[...], axis=0)[:, None]
        h = h + jnp.dot(cont_ref[...], w1c_ref[...],
                        preferred_element_type=jnp.float32)
        h = jnp.maximum(h + b1_ref[...][None, :], 0.0)
        h = _ln(h, g1_ref[...], be1_ref[...])
        h = jnp.dot(h.astype(jnp.bfloat16), w2_ref[...],
                    preferred_element_type=jnp.float32)
        h = jnp.maximum(h + b2_ref[...][None, :], 0.0)
        h = _ln(h, g2_ref[...], be2_ref[...])
        h = jnp.dot(h.astype(jnp.bfloat16), w3_ref[...],
                    preferred_element_type=jnp.float32)
        h = jnp.maximum(h + b3_ref[...][None, :], 0.0)
        h = _ln(h, g3_ref[...], be3_ref[...])
        deep = jnp.dot(h, w4_ref[...], preferred_element_type=jnp.float32)
        out_ref[...] = first + second + deep + b4_ref[0]

    return body


def _tc_mlp_part(x3, cont, lv, w1e, w1c, b1, g1, be1, w2, b2, g2, be2, w3,
                 b3, g3, be3, w4, b4, part, emb_prev):
    BB = 512
    grid = (PB // BB,)
    nb0 = part * (PB // BB)
    row = lambda i: (i, 0)
    rep2 = lambda i: (0, 0)
    rep1 = lambda i: (0,)
    h1, h2, h3 = 1024, 512, 256
    in_specs = [
        pl.BlockSpec((F, BB, D), lambda i: (0, i, 0)),
        pl.BlockSpec((BB, NCF), lambda i: (nb0 + i, 0)),
        pl.BlockSpec((F, BB), lambda i: (0, i)),
        pl.BlockSpec((F * D, h1), rep2),
        pl.BlockSpec((NCF, h1), rep2),
        pl.BlockSpec((h1,), rep1),
        pl.BlockSpec((h1,), rep1),
        pl.BlockSpec((h1,), rep1),
        pl.BlockSpec((h1, h2), rep2),
        pl.BlockSpec((h2,), rep1),
        pl.BlockSpec((h2,), rep1),
        pl.BlockSpec((h2,), rep1),
        pl.BlockSpec((h2, h3), rep2),
        pl.BlockSpec((h3,), rep1),
        pl.BlockSpec((h3,), rep1),
        pl.BlockSpec((h3,), rep1),
        pl.BlockSpec((h3, 1), rep2),
        pl.BlockSpec((1,), rep1),
    ]
    args = [x3, cont, lv, w1e, w1c, b1, g1, be1, w2, b2, g2, be2, w3, b3, g3,
            be3, w4, b4]
    aliases = {}
    if emb_prev is not None:
        in_specs.append(pl.BlockSpec(memory_space=pl.ANY))
        args.append(emb_prev)
        aliases = {18: 1}
    return pl.pallas_call(
        _make_tc_body(emb_prev is not None),
        grid=grid,
        in_specs=in_specs,
        out_specs=[
            pl.BlockSpec((BB, 1), row),
            pl.BlockSpec((F, BB, D), lambda i: (0, nb0 + i, 0)),
        ],
        out_shape=[
            jax.ShapeDtypeStruct((PB, 1), jnp.float32),
            jax.ShapeDtypeStruct((F, B, D), jnp.float32),
        ],
        input_output_aliases=aliases,
        compiler_params=pltpu.CompilerParams(
            dimension_semantics=("arbitrary",)),
    )(*args)


def kernel(field_indices, continuous_features, embedding, linear_emb, W1, b1,
           g1, be1, W2, b2, g2, be2, W3, b3, g3, be3, W4, b4):
    ft = field_indices.astype(jnp.int32).T
    lin_flat = linear_emb.reshape(-1)
    fitp, rows, lvs = [], [], []
    for p in range(NPART):
        fitp.append(ft[:, p * PB:(p + 1) * PB].reshape(PROWS // 128, 128))
    for p in range(NPART):
        lvs.append(_sc_linear_part(fitp[p].reshape(-1), lin_flat))
        rows.append(_sc_gather_part(fitp[p], embedding))
    w1e = W1[:F * D].astype(jnp.bfloat16)
    w1c = W1[F * D:]
    w2b = W2.astype(jnp.bfloat16)
    w3b = W3.astype(jnp.bfloat16)
    logits_parts = []
    emb = None
    for p in range(NPART):
        x3 = rows[p].reshape(F, PB, D)
        lp, emb = _tc_mlp_part(x3, continuous_features, lvs[p], w1e, w1c, b1,
                               g1, be1, w2b, b2, g2, be2, w3b, b3, g3, be3,
                               W4, b4, p, emb)
        logits_parts.append(lp)
    logits = jnp.concatenate(logits_parts, axis=0)
    embeds = emb.transpose(1, 0, 2)
    return (logits, embeds)


# speedup vs baseline: 1.1051x; 1.1051x over previous
import functools

import jax
import jax.numpy as jnp
from jax import lax
from jax.experimental import pallas as pl
from jax.experimental.pallas import tpu as pltpu
from jax.experimental.pallas import tpu_sc as plsc

B = 16384
F = 26
V = 1000
D = 128
NCF = 4
ROWS = B * F
NPART = 2
PB = B // NPART
LOG2PB = 13
PROWS = PB * F

SC_CORES = 2
SC_SUBCORES = 16
NW = SC_CORES * SC_SUBCORES
PROWS_PER_W = PROWS // NW
CH = 512
CHB = CH // 128
PNCHUNK = PROWS_PER_W // CH


def _sc_gather_part(fitp, emb):
    mesh = plsc.VectorSubcoreMesh(core_axis_name="c", subcore_axis_name="s")

    @functools.partial(
        pl.kernel,
        mesh=mesh,
        out_type=jax.ShapeDtypeStruct((PROWS, D), jnp.float32),
        scratch_types=[
            pltpu.VMEM((CHB, 128), jnp.int32),
            pltpu.VMEM((CH, D), jnp.float32),
            pltpu.SemaphoreType.DMA,
        ],
    )
    def k(fi_hbm, emb_hbm, oute_hbm, idx_v, rows_v, sem_e):
        wid = lax.axis_index("c") * SC_SUBCORES + lax.axis_index("s")
        w_base = wid * PROWS_PER_W
        w_row0 = wid * (PROWS_PER_W // 128)

        def chunk_body(ci, carry):
            qp0 = w_base + ci * CH
            fld = lax.shift_right_logical(qp0, LOG2PB)
            rb = w_row0 + ci * CHB
            pltpu.sync_copy(fi_hbm.at[pl.ds(rb, CHB)], idx_v)
            off = fld * V
            for j in range(CHB):
                for k16 in range(8):
                    sl = (j, pl.ds(k16 * 16, 16))
                    idx_v[sl] = idx_v[sl] + off
            copies = [
                pltpu.make_async_copy(
                    emb_hbm.at[idx_v.at[j]],
                    rows_v.at[pl.ds(j * 128, 128)], sem_e)
                for j in range(CHB)
            ]
            for c in copies:
                c.start()
            for c in copies:
                c.wait()
            pltpu.sync_copy(rows_v, oute_hbm.at[pl.ds(qp0, CH)])
            return carry

        lax.fori_loop(0, PNCHUNK, chunk_body, 0)

    return k(fitp, emb)


LCH = 512
LNCHUNK = PROWS_PER_W // LCH


def _sc_linear_part(fi_flat, lin_flat):
    mesh = plsc.VectorSubcoreMesh(core_axis_name="c", subcore_axis_name="s")

    @functools.partial(
        pl.kernel,
        mesh=mesh,
        out_type=jax.ShapeDtypeStruct((F, PB), jnp.float32),
        scratch_types=[
            pltpu.VMEM((F * V,), jnp.float32),
            pltpu.VMEM((LCH,), jnp.int32),
            pltpu.VMEM((LCH,), jnp.float32),
        ],
        compiler_params=pltpu.CompilerParams(needs_layout_passes=False),
    )
    def k(fi_hbm, lin_hbm, outl_hbm, tab_v, idx_v, val_v):
        wid = lax.axis_index("c") * SC_SUBCORES + lax.axis_index("s")
        w_base = wid * PROWS_PER_W
        pltpu.sync_copy(lin_hbm, tab_v)

        def chunk_body(ci, carry):
            base = w_base + ci * LCH
            fld = lax.shift_right_logical(base, LOG2PB)
            s0 = base - fld * PB
            pltpu.sync_copy(fi_hbm.at[pl.ds(base, LCH)], idx_v)
            off = fld * V
            for j in range(LCH // 16):
                sl = pl.ds(j * 16, 16)
                val_v[sl] = plsc.load_gather(tab_v, [idx_v[sl] + off])
            pltpu.sync_copy(val_v, outl_hbm.at[fld, pl.ds(s0, LCH)])
            return carry

        lax.fori_loop(0, LNCHUNK, chunk_body, 0)

    return k(fi_flat, lin_flat)


def _ln(h, g, b):
    m = jnp.mean(h, axis=-1, keepdims=True)
    d = h - m
    v = jnp.mean(d * d, axis=-1, keepdims=True)
    return d * lax.rsqrt(v + 1e-5) * g[None, :] + b[None, :]


def _make_tc_body(with_prev):
    def body(x3_ref, cont_ref, lv_ref, w1e_ref, w1c_ref, b1_ref, g1_ref,
             be1_ref, w2_ref, b2_ref, g2_ref, be2_ref, w3_ref, b3_ref,
             g3_ref, be3_ref, w4_ref, b4_ref, *rest):
        if with_prev:
            _, out_ref, embout_ref = rest
        else:
            out_ref, embout_ref = rest
        x3 = x3_ref[...]
        embout_ref[...] = x3
        xf = x3[0]
        s = xf
        ss = xf * xf
        for f in range(1, F):
            xf = x3[f]
            s = s + xf
            ss = ss + xf * xf
        xall = jnp.concatenate([x3[f] for f in range(F)], axis=1)
        h = jnp.dot(xall.astype(jnp.bfloat16), w1e_ref[...],
                    preferred_element_type=jnp.float32)
        second = 0.5 * jnp.sum(s * s - ss, axis=1, keepdims=True)
        first = jnp.sum(lv_ref[...], axis=0)[:, None]
        h = h + jnp.dot(cont_ref[...], w1c_ref[...],
                        preferred_element_type=jnp.float32)
        h = jnp.maximum(h + b1_ref[...][None, :], 0.0)
        h = _ln(h, g1_ref[...], be1_ref[...])
        h = jnp.dot(h.astype(jnp.bfloat16), w2_ref[...],
                    preferred_element_type=jnp.float32)
        h = jnp.maximum(h + b2_ref[...][None, :], 0.0)
        h = _ln(h, g2_ref[...], be2_ref[...])
        h = jnp.dot(h.astype(jnp.bfloat16), w3_ref[...],
                    preferred_element_type=jnp.float32)
        h = jnp.maximum(h + b3_ref[...][None, :], 0.0)
        h = _ln(h, g3_ref[...], be3_ref[...])
        deep = jnp.dot(h, w4_ref[...], preferred_element_type=jnp.float32)
        out_ref[...] = first + second + deep + b4_ref[0]

    return body


def _tc_mlp_part(x3, cont, lv, w1e, w1c, b1, g1, be1, w2, b2, g2, be2, w3,
                 b3, g3, be3, w4, b4, part, emb_prev):
    BB = 512
    grid = (PB // BB,)
    nb0 = part * (PB // BB)
    row = lambda i: (i, 0)
    rep2 = lambda i: (0, 0)
    rep1 = lambda i: (0,)
    h1, h2, h3 = 1024, 512, 256
    in_specs = [
        pl.BlockSpec((F, BB, D), lambda i: (0, i, 0)),
        pl.BlockSpec((BB, NCF), lambda i: (nb0 + i, 0)),
        pl.BlockSpec((F, BB), lambda i: (0, i)),
        pl.BlockSpec((F * D, h1), rep2),
        pl.BlockSpec((NCF, h1), rep2),
        pl.BlockSpec((h1,), rep1),
        pl.BlockSpec((h1,), rep1),
        pl.BlockSpec((h1,), rep1),
        pl.BlockSpec((h1, h2), rep2),
        pl.BlockSpec((h2,), rep1),
        pl.BlockSpec((h2,), rep1),
        pl.BlockSpec((h2,), rep1),
        pl.BlockSpec((h2, h3), rep2),
        pl.BlockSpec((h3,), rep1),
        pl.BlockSpec((h3,), rep1),
        pl.BlockSpec((h3,), rep1),
        pl.BlockSpec((h3, 1), rep2),
        pl.BlockSpec((1,), rep1),
    ]
    args = [x3, cont, lv, w1e, w1c, b1, g1, be1, w2, b2, g2, be2, w3, b3, g3,
            be3, w4, b4]
    aliases = {}
    if emb_prev is not None:
        in_specs.append(pl.BlockSpec(memory_space=pl.ANY))
        args.append(emb_prev)
        aliases = {18: 1}
    return pl.pallas_call(
        _make_tc_body(emb_prev is not None),
        grid=grid,
        in_specs=in_specs,
        out_specs=[
            pl.BlockSpec((BB, 1), row),
            pl.BlockSpec((F, BB, D), lambda i: (0, nb0 + i, 0)),
        ],
        out_shape=[
            jax.ShapeDtypeStruct((PB, 1), jnp.float32),
            jax.ShapeDtypeStruct((F, B, D), jnp.float32),
        ],
        input_output_aliases=aliases,
        compiler_params=pltpu.CompilerParams(
            dimension_semantics=("arbitrary",)),
    )(*args)


def kernel(field_indices, continuous_features, embedding, linear_emb, W1, b1,
           g1, be1, W2, b2, g2, be2, W3, b3, g3, be3, W4, b4):
    ft = field_indices.astype(jnp.int32).T
    lin_flat = linear_emb.reshape(-1)
    fitp, rows, lvs = [], [], []
    for p in range(NPART):
        fitp.append(ft[:, p * PB:(p + 1) * PB].reshape(PROWS // 128, 128))
    for p in range(NPART):
        lvs.append(_sc_linear_part(fitp[p].reshape(-1), lin_flat))
        rows.append(_sc_gather_part(fitp[p], embedding))
    w1e = W1[:F * D].astype(jnp.bfloat16)
    w1c = W1[F * D:]
    w2b = W2.astype(jnp.bfloat16)
    w3b = W3.astype(jnp.bfloat16)
    logits_parts = []
    emb = None
    for p in range(NPART):
        x3 = rows[p].reshape(F, PB, D)
        lp, emb = _tc_mlp_part(x3, continuous_features, lvs[p], w1e, w1c, b1,
                               g1, be1, w2b, b2, g2, be2, w3b, b3, g3, be3,
                               W4, b4, p, emb)
        logits_parts.append(lp)
    logits = jnp.concatenate(logits_parts, axis=0)
    embeds = emb.transpose(1, 0, 2)
    return (logits, embeds)


# speedup vs baseline: 1.1288x; 1.0214x over previous
import functools

import jax
import jax.numpy as jnp
from jax import lax
from jax.experimental import pallas as pl
from jax.experimental.pallas import tpu as pltpu
from jax.experimental.pallas import tpu_sc as plsc

B = 16384
F = 26
V = 1000
D = 128
NCF = 4
ROWS = B * F
NPART = 2
PB = B // NPART
LOG2PB = 13
PROWS = PB * F

SC_CORES = 2
SC_SUBCORES = 16
NW = SC_CORES * SC_SUBCORES
PROWS_PER_W = PROWS // NW
CH = 512
CHB = CH // 128
PNCHUNK = PROWS_PER_W // CH


def _sc_gather_part(fitp, emb):
    mesh = plsc.VectorSubcoreMesh(core_axis_name="c", subcore_axis_name="s")

    @functools.partial(
        pl.kernel,
        mesh=mesh,
        out_type=jax.ShapeDtypeStruct((PROWS, D), jnp.float32),
        scratch_types=[
            pltpu.VMEM((CHB, 128), jnp.int32),
            pltpu.VMEM((CH, D), jnp.float32),
            pltpu.SemaphoreType.DMA,
        ],
    )
    def k(fi_hbm, emb_hbm, oute_hbm, idx_v, rows_v, sem_e):
        wid = lax.axis_index("c") * SC_SUBCORES + lax.axis_index("s")
        w_base = wid * PROWS_PER_W
        w_row0 = wid * (PROWS_PER_W // 128)

        def chunk_body(ci, carry):
            qp0 = w_base + ci * CH
            fld = lax.shift_right_logical(qp0, LOG2PB)
            rb = w_row0 + ci * CHB
            pltpu.sync_copy(fi_hbm.at[pl.ds(rb, CHB)], idx_v)
            off = fld * V
            for j in range(CHB):
                for k16 in range(8):
                    sl = (j, pl.ds(k16 * 16, 16))
                    idx_v[sl] = idx_v[sl] + off
            copies = [
                pltpu.make_async_copy(
                    emb_hbm.at[idx_v.at[j]],
                    rows_v.at[pl.ds(j * 128, 128)], sem_e)
                for j in range(CHB)
            ]
            for c in copies:
                c.start()
            for c in copies:
                c.wait()
            pltpu.sync_copy(rows_v, oute_hbm.at[pl.ds(qp0, CH)])
            return carry

        lax.fori_loop(0, PNCHUNK, chunk_body, 0)

    return k(fitp, emb)


LCH = 512


LROWS_PER_W = ROWS // NW
LNCHUNK_FULL = LROWS_PER_W // LCH
LOG2B = 14


def _sc_linear(fi_flat, lin_flat):
    mesh = plsc.VectorSubcoreMesh(core_axis_name="c", subcore_axis_name="s")

    @functools.partial(
        pl.kernel,
        mesh=mesh,
        out_type=jax.ShapeDtypeStruct((F, B), jnp.float32),
        scratch_types=[
            pltpu.VMEM((F * V,), jnp.float32),
            pltpu.VMEM((LCH,), jnp.int32),
            pltpu.VMEM((LCH,), jnp.float32),
        ],
        compiler_params=pltpu.CompilerParams(needs_layout_passes=False),
    )
    def k(fi_hbm, lin_hbm, outl_hbm, tab_v, idx_v, val_v):
        wid = lax.axis_index("c") * SC_SUBCORES + lax.axis_index("s")
        w_base = wid * LROWS_PER_W
        pltpu.sync_copy(lin_hbm, tab_v)

        def chunk_body(ci, carry):
            base = w_base + ci * LCH
            fld = lax.shift_right_logical(base, LOG2B)
            s0 = base - fld * B
            pltpu.sync_copy(fi_hbm.at[pl.ds(base, LCH)], idx_v)
            off = fld * V
            for j in range(LCH // 16):
                sl = pl.ds(j * 16, 16)
                val_v[sl] = plsc.load_gather(tab_v, [idx_v[sl] + off])
            pltpu.sync_copy(val_v, outl_hbm.at[fld, pl.ds(s0, LCH)])
            return carry

        lax.fori_loop(0, LNCHUNK_FULL, chunk_body, 0)

    return k(fi_flat, lin_flat)


def _ln(h, g, b):
    m = jnp.mean(h, axis=-1, keepdims=True)
    d = h - m
    v = jnp.mean(d * d, axis=-1, keepdims=True)
    return d * lax.rsqrt(v + 1e-5) * g[None, :] + b[None, :]


def _make_tc_body(with_prev):
    def body(x3_ref, cont_ref, lv_ref, w1e_ref, w1c_ref, b1_ref, g1_ref,
             be1_ref, w2_ref, b2_ref, g2_ref, be2_ref, w3_ref, b3_ref,
             g3_ref, be3_ref, w4_ref, b4_ref, *rest):
        if with_prev:
            _, out_ref, embout_ref = rest
        else:
            out_ref, embout_ref = rest
        x3 = x3_ref[...]
        embout_ref[...] = x3
        xf = x3[0]
        s = xf
        ss = xf * xf
        for f in range(1, F):
            xf = x3[f]
            s = s + xf
            ss = ss + xf * xf
        xall = jnp.concatenate([x3[f] for f in range(F)], axis=1)
        h = jnp.dot(xall.astype(jnp.bfloat16), w1e_ref[...],
                    preferred_element_type=jnp.float32)
        second = 0.5 * jnp.sum(s * s - ss, axis=1, keepdims=True)
        first = jnp.sum(lv_ref[...], axis=0)[:, None]
        h = h + jnp.dot(cont_ref[...], w1c_ref[...],
                        preferred_element_type=jnp.float32)
        h = jnp.maximum(h + b1_ref[...][None, :], 0.0)
        h = _ln(h, g1_ref[...], be1_ref[...])
        h = jnp.dot(h.astype(jnp.bfloat16), w2_ref[...],
                    preferred_element_type=jnp.float32)
        h = jnp.maximum(h + b2_ref[...][None, :], 0.0)
        h = _ln(h, g2_ref[...], be2_ref[...])
        h = jnp.dot(h.astype(jnp.bfloat16), w3_ref[...],
                    preferred_element_type=jnp.float32)
        h = jnp.maximum(h + b3_ref[...][None, :], 0.0)
        h = _ln(h, g3_ref[...], be3_ref[...])
        deep = jnp.dot(h, w4_ref[...], preferred_element_type=jnp.float32)
        out_ref[...] = first + second + deep + b4_ref[0]

    return body


def _tc_mlp_part(x3, cont, lv, w1e, w1c, b1, g1, be1, w2, b2, g2, be2, w3,
                 b3, g3, be3, w4, b4, part, emb_prev):
    BB = 512
    grid = (PB // BB,)
    nb0 = part * (PB // BB)
    row = lambda i: (i, 0)
    rep2 = lambda i: (0, 0)
    rep1 = lambda i: (0,)
    h1, h2, h3 = 1024, 512, 256
    in_specs = [
        pl.BlockSpec((F, BB, D), lambda i: (0, i, 0)),
        pl.BlockSpec((BB, NCF), lambda i: (nb0 + i, 0)),
        pl.BlockSpec((F, BB), lambda i: (0, nb0 + i)),
        pl.BlockSpec((F * D, h1), rep2),
        pl.BlockSpec((NCF, h1), rep2),
        pl.BlockSpec((h1,), rep1),
        pl.BlockSpec((h1,), rep1),
        pl.BlockSpec((h1,), rep1),
        pl.BlockSpec((h1, h2), rep2),
        pl.BlockSpec((h2,), rep1),
        pl.BlockSpec((h2,), rep1),
        pl.BlockSpec((h2,), rep1),
        pl.BlockSpec((h2, h3), rep2),
        pl.BlockSpec((h3,), rep1),
        pl.BlockSpec((h3,), rep1),
        pl.BlockSpec((h3,), rep1),
        pl.BlockSpec((h3, 1), rep2),
        pl.BlockSpec((1,), rep1),
    ]
    args = [x3, cont, lv, w1e, w1c, b1, g1, be1, w2, b2, g2, be2, w3, b3, g3,
            be3, w4, b4]
    aliases = {}
    if emb_prev is not None:
        in_specs.append(pl.BlockSpec(memory_space=pl.ANY))
        args.append(emb_prev)
        aliases = {18: 1}
    return pl.pallas_call(
        _make_tc_body(emb_prev is not None),
        grid=grid,
        in_specs=in_specs,
        out_specs=[
            pl.BlockSpec((BB, 1), row),
            pl.BlockSpec((F, BB, D), lambda i: (0, nb0 + i, 0)),
        ],
        out_shape=[
            jax.ShapeDtypeStruct((PB, 1), jnp.float32),
            jax.ShapeDtypeStruct((F, B, D), jnp.float32),
        ],
        input_output_aliases=aliases,
        compiler_params=pltpu.CompilerParams(
            dimension_semantics=("arbitrary",)),
    )(*args)


def kernel(field_indices, continuous_features, embedding, linear_emb, W1, b1,
           g1, be1, W2, b2, g2, be2, W3, b3, g3, be3, W4, b4):
    ft = field_indices.astype(jnp.int32).T
    lin_flat = linear_emb.reshape(-1)
    fitp, rows = [], []
    for p in range(NPART):
        fitp.append(ft[:, p * PB:(p + 1) * PB].reshape(PROWS // 128, 128))
    lv = _sc_linear(ft.reshape(-1), lin_flat)
    for p in range(NPART):
        rows.append(_sc_gather_part(fitp[p], embedding))
    w1e = W1[:F * D].astype(jnp.bfloat16)
    w1c = W1[F * D:]
    w2b = W2.astype(jnp.bfloat16)
    w3b = W3.astype(jnp.bfloat16)
    logits_parts = []
    emb = None
    for p in range(NPART):
        x3 = rows[p].reshape(F, PB, D)
        lp, emb = _tc_mlp_part(x3, continuous_features, lv, w1e, w1c, b1,
                               g1, be1, w2b, b2, g2, be2, w3b, b3, g3, be3,
                               W4, b4, p, emb)
        logits_parts.append(lp)
    logits = jnp.concatenate(logits_parts, axis=0)
    embeds = emb.transpose(1, 0, 2)
    return (logits, embeds)


# speedup vs baseline: 1.1465x; 1.0157x over previous
import functools

import jax
import jax.numpy as jnp
from jax import lax
from jax.experimental import pallas as pl
from jax.experimental.pallas import tpu as pltpu
from jax.experimental.pallas import tpu_sc as plsc

B = 16384
F = 26
V = 1000
D = 128
NCF = 4
ROWS = B * F
NPART = 2
PB = B // NPART
LOG2PB = 13
PROWS = PB * F

SC_CORES = 2
SC_SUBCORES = 16
NW = SC_CORES * SC_SUBCORES
PROWS_PER_W = PROWS // NW
CH = 256
CHB = CH // 128
PNCHUNK = PROWS_PER_W // CH


def _sc_gather_part(fitp, emb):
    mesh = plsc.VectorSubcoreMesh(core_axis_name="c", subcore_axis_name="s")

    @functools.partial(
        pl.kernel,
        mesh=mesh,
        out_type=jax.ShapeDtypeStruct((PROWS, D), jnp.float32),
        scratch_types=[
            pltpu.VMEM((CHB, 128), jnp.int32),
            pltpu.VMEM((CHB, 128), jnp.int32),
            pltpu.VMEM((CH, D), jnp.float32),
            pltpu.VMEM((CH, D), jnp.float32),
            pltpu.SemaphoreType.DMA,
            pltpu.SemaphoreType.DMA,
            pltpu.SemaphoreType.DMA,
            pltpu.SemaphoreType.DMA,
        ],
    )
    def k(fi_hbm, emb_hbm, oute_hbm, i0, i1, r0, r1, e0, e1, w0, w1):
        wid = lax.axis_index("c") * SC_SUBCORES + lax.axis_index("s")
        w_base = wid * PROWS_PER_W
        w_row0 = wid * (PROWS_PER_W // 128)

        def stage(ci, idx_v):
            rb = w_row0 + ci * CHB
            pltpu.sync_copy(fi_hbm.at[pl.ds(rb, CHB)], idx_v)
            off = lax.shift_right_logical(w_base + ci * CH, LOG2PB) * V
            for j in range(CHB):
                for k16 in range(8):
                    sl = (j, pl.ds(k16 * 16, 16))
                    idx_v[sl] = idx_v[sl] + off

        def gstart(idx_v, rows_v, sem):
            for j in range(CHB):
                pltpu.make_async_copy(
                    emb_hbm.at[idx_v.at[j]],
                    rows_v.at[pl.ds(j * 128, 128)], sem).start()

        def gwait(idx_v, rows_v, sem):
            for j in range(CHB):
                pltpu.make_async_copy(
                    emb_hbm.at[idx_v.at[j]],
                    rows_v.at[pl.ds(j * 128, 128)], sem).wait()

        def sstart(ci, rows_v, sem):
            pltpu.make_async_copy(
                rows_v, oute_hbm.at[pl.ds(w_base + ci * CH, CH)], sem).start()

        def swait(ci, rows_v, sem):
            pltpu.make_async_copy(
                rows_v, oute_hbm.at[pl.ds(w_base + ci * CH, CH)], sem).wait()

        stage(0, i0)
        gstart(i0, r0, e0)
        stage(1, i1)
        gstart(i1, r1, e1)

        def body(g, carry):
            a = 2 * g
            b = a + 1
            gwait(i0, r0, e0)
            sstart(a, r0, w0)
            gwait(i1, r1, e1)
            sstart(b, r1, w1)
            stage(a + 2, i0)
            swait(a, r0, w0)
            gstart(i0, r0, e0)
            stage(b + 2, i1)
            swait(b, r1, w1)
            gstart(i1, r1, e1)
            return carry

        lax.fori_loop(0, PNCHUNK // 2 - 1, body, 0)
        gwait(i0, r0, e0)
        sstart(PNCHUNK - 2, r0, w0)
        gwait(i1, r1, e1)
        sstart(PNCHUNK - 1, r1, w1)
        swait(PNCHUNK - 2, r0, w0)
        swait(PNCHUNK - 1, r1, w1)

    return k(fitp, emb)


LCH = 512


LROWS_PER_W = ROWS // NW
LNCHUNK_FULL = LROWS_PER_W // LCH
LOG2B = 14


def _sc_linear(fi_flat, lin_flat):
    mesh = plsc.VectorSubcoreMesh(core_axis_name="c", subcore_axis_name="s")

    @functools.partial(
        pl.kernel,
        mesh=mesh,
        out_type=jax.ShapeDtypeStruct((F, B), jnp.float32),
        scratch_types=[
            pltpu.VMEM((F * V,), jnp.float32),
            pltpu.VMEM((LCH,), jnp.int32),
            pltpu.VMEM((LCH,), jnp.float32),
        ],
        compiler_params=pltpu.CompilerParams(needs_layout_passes=False),
    )
    def k(fi_hbm, lin_hbm, outl_hbm, tab_v, idx_v, val_v):
        wid = lax.axis_index("c") * SC_SUBCORES + lax.axis_index("s")
        w_base = wid * LROWS_PER_W
        pltpu.sync_copy(lin_hbm, tab_v)

        def chunk_body(ci, carry):
            base = w_base + ci * LCH
            fld = lax.shift_right_logical(base, LOG2B)
            s0 = base - fld * B
            pltpu.sync_copy(fi_hbm.at[pl.ds(base, LCH)], idx_v)
            off = fld * V
            for j in range(LCH // 16):
                sl = pl.ds(j * 16, 16)
                val_v[sl] = plsc.load_gather(tab_v, [idx_v[sl] + off])
            pltpu.sync_copy(val_v, outl_hbm.at[fld, pl.ds(s0, LCH)])
            return carry

        lax.fori_loop(0, LNCHUNK_FULL, chunk_body, 0)

    return k(fi_flat, lin_flat)


def _ln(h, g, b):
    m = jnp.mean(h, axis=-1, keepdims=True)
    d = h - m
    v = jnp.mean(d * d, axis=-1, keepdims=True)
    return d * lax.rsqrt(v + 1e-5) * g[None, :] + b[None, :]


def _make_tc_body(with_prev):
    def body(x3_ref, cont_ref, lv_ref, w1e_ref, w1c_ref, b1_ref, g1_ref,
             be1_ref, w2_ref, b2_ref, g2_ref, be2_ref, w3_ref, b3_ref,
             g3_ref, be3_ref, w4_ref, b4_ref, *rest):
        if with_prev:
            _, out_ref, embout_ref = rest
        else:
            out_ref, embout_ref = rest
        x3 = x3_ref[...]
        embout_ref[...] = x3
        xf = x3[0]
        s = xf
        ss = xf * xf
        for f in range(1, F):
            xf = x3[f]
            s = s + xf
            ss = ss + xf * xf
        xall = jnp.concatenate([x3[f] for f in range(F)], axis=1)
        h = jnp.dot(xall.astype(jnp.bfloat16), w1e_ref[...],
                    preferred_element_type=jnp.float32)
        second = 0.5 * jnp.sum(s * s - ss, axis=1, keepdims=True)
        first = jnp.sum(lv_ref[...], axis=0)[:, None]
        h = h + jnp.dot(cont_ref[...], w1c_ref[...],
                        preferred_element_type=jnp.float32)
        h = jnp.maximum(h + b1_ref[...][None, :], 0.0)
        h = _ln(h, g1_ref[...], be1_ref[...])
        h = jnp.dot(h.astype(jnp.bfloat16), w2_ref[...],
                    preferred_element_type=jnp.float32)
        h = jnp.maximum(h + b2_ref[...][None, :], 0.0)
        h = _ln(h, g2_ref[...], be2_ref[...])
        h = jnp.dot(h.astype(jnp.bfloat16), w3_ref[...],
                    preferred_element_type=jnp.float32)
        h = jnp.maximum(h + b3_ref[...][None, :], 0.0)
        h = _ln(h, g3_ref[...], be3_ref[...])
        deep = jnp.dot(h, w4_ref[...], preferred_element_type=jnp.float32)
        out_ref[...] = first + second + deep + b4_ref[0]

    return body


def _tc_mlp_part(x3, cont, lv, w1e, w1c, b1, g1, be1, w2, b2, g2, be2, w3,
                 b3, g3, be3, w4, b4, part, emb_prev):
    BB = 512
    grid = (PB // BB,)
    nb0 = part * (PB // BB)
    row = lambda i: (i, 0)
    rep2 = lambda i: (0, 0)
    rep1 = lambda i: (0,)
    h1, h2, h3 = 1024, 512, 256
    in_specs = [
        pl.BlockSpec((F, BB, D), lambda i: (0, i, 0)),
        pl.BlockSpec((BB, NCF), lambda i: (nb0 + i, 0)),
        pl.BlockSpec((F, BB), lambda i: (0, nb0 + i)),
        pl.BlockSpec((F * D, h1), rep2),
        pl.BlockSpec((NCF, h1), rep2),
        pl.BlockSpec((h1,), rep1),
        pl.BlockSpec((h1,), rep1),
        pl.BlockSpec((h1,), rep1),
        pl.BlockSpec((h1, h2), rep2),
        pl.BlockSpec((h2,), rep1),
        pl.BlockSpec((h2,), rep1),
        pl.BlockSpec((h2,), rep1),
        pl.BlockSpec((h2, h3), rep2),
        pl.BlockSpec((h3,), rep1),
        pl.BlockSpec((h3,), rep1),
        pl.BlockSpec((h3,), rep1),
        pl.BlockSpec((h3, 1), rep2),
        pl.BlockSpec((1,), rep1),
    ]
    args = [x3, cont, lv, w1e, w1c, b1, g1, be1, w2, b2, g2, be2, w3, b3, g3,
            be3, w4, b4]
    aliases = {}
    if emb_prev is not None:
        in_specs.append(pl.BlockSpec(memory_space=pl.ANY))
        args.append(emb_prev)
        aliases = {18: 1}
    return pl.pallas_call(
        _make_tc_body(emb_prev is not None),
        grid=grid,
        in_specs=in_specs,
        out_specs=[
            pl.BlockSpec((BB, 1), row),
            pl.BlockSpec((F, BB, D), lambda i: (0, nb0 + i, 0)),
        ],
        out_shape=[
            jax.ShapeDtypeStruct((PB, 1), jnp.float32),
            jax.ShapeDtypeStruct((F, B, D), jnp.float32),
        ],
        input_output_aliases=aliases,
        compiler_params=pltpu.CompilerParams(
            dimension_semantics=("arbitrary",)),
    )(*args)


def kernel(field_indices, continuous_features, embedding, linear_emb, W1, b1,
           g1, be1, W2, b2, g2, be2, W3, b3, g3, be3, W4, b4):
    ft = field_indices.astype(jnp.int32).T
    lin_flat = linear_emb.reshape(-1)
    fitp, rows = [], []
    for p in range(NPART):
        fitp.append(ft[:, p * PB:(p + 1) * PB].reshape(PROWS // 128, 128))
    lv = _sc_linear(ft.reshape(-1), lin_flat)
    for p in range(NPART):
        rows.append(_sc_gather_part(fitp[p], embedding))
    w1e = W1[:F * D].astype(jnp.bfloat16)
    w1c = W1[F * D:]
    w2b = W2.astype(jnp.bfloat16)
    w3b = W3.astype(jnp.bfloat16)
    logits_parts = []
    emb = None
    for p in range(NPART):
        x3 = rows[p].reshape(F, PB, D)
        lp, emb = _tc_mlp_part(x3, continuous_features, lv, w1e, w1c, b1,
                               g1, be1, w2b, b2, g2, be2, w3b, b3, g3, be3,
                               W4, b4, p, emb)
        logits_parts.append(lp)
    logits = jnp.concatenate(logits_parts, axis=0)
    embeds = emb.transpose(1, 0, 2)
    return (logits, embeds)
